# Initial kernel scaffold; baseline (speedup 1.0000x reference)
import jax
import jax.numpy as jnp
from jax.experimental import pallas as pl


def kernel(feats, W):
    raise NotImplementedError("write your pallas kernel here")


# speedup vs baseline: 1.1944x; 1.1944x over previous
import functools

import jax
import jax.numpy as jnp
from jax import lax
from jax.experimental import pallas as pl
from jax.experimental.pallas import tpu as pltpu
from jax.experimental.pallas import tpu_sc as plsc

K = 8192
D = 256
M = 16384

BM = 1024
BK = 1024
NM = M // BM
NK = K // BK


def _dist_argmin_body(f_ref, w_ref, idx_ref, loss_ref, minv, mini, fnv, acc):
    m = pl.program_id(0)
    k = pl.program_id(1)

    @pl.when(k == 0)
    def _init():
        f = f_ref[...]
        fnv[...] = jnp.sum(f * f, axis=1, keepdims=True)
        minv[...] = jnp.full((BM, 1), jnp.inf, jnp.float32)
        mini[...] = jnp.zeros((BM, 1), jnp.int32)

    w = w_ref[...]
    wn = jnp.sum(w * w, axis=1)
    d = lax.dot_general(f_ref[...], w, (((1,), (1,)), ((), ())),
                        preferred_element_type=jnp.float32)
    s = (fnv[...] - 2.0 * d) + wn[None, :]
    bmin = jnp.min(s, axis=1, keepdims=True)
    cand = jnp.where(s == bmin,
                     lax.broadcasted_iota(jnp.int32, (BM, BK), 1),
                     jnp.int32(BK))
    barg = jnp.min(cand, axis=1, keepdims=True) + k * BK
    better = bmin < minv[...]
    mini[...] = jnp.where(better, barg, mini[...])
    minv[...] = jnp.where(better, bmin, minv[...])

    @pl.when(k == NK - 1)
    def _finish():
        idx_ref[0, 0, :] = mini[:, 0]

        @pl.when(m == 0)
        def _zero():
            acc[0] = 0.0

        acc[0] += jnp.sum(minv[...])

        @pl.when(m == NM - 1)
        def _loss():
            loss_ref[...] = jnp.full((1, 1), acc[0] * (1.25 / (M * D)),
                                     jnp.float32)


def _dist_argmin(flat, W):
    idx3, loss = pl.pallas_call(
        _dist_argmin_body,
        grid=(NM, NK),
        in_specs=[
            pl.BlockSpec((BM, D), lambda m, k: (m, 0)),
            pl.BlockSpec((BK, D), lambda m, k: (k, 0)),
        ],
        out_specs=[
            pl.BlockSpec((1, 1, BM), lambda m, k: (m, 0, 0)),
            pl.BlockSpec((1, 1), lambda m, k: (0, 0)),
        ],
        out_shape=[
            jax.ShapeDtypeStruct((NM, 1, BM), jnp.int32),
            jax.ShapeDtypeStruct((1, 1), jnp.float32),
        ],
        scratch_shapes=[
            pltpu.VMEM((BM, 1), jnp.float32),
            pltpu.VMEM((BM, 1), jnp.int32),
            pltpu.VMEM((BM, 1), jnp.float32),
            pltpu.SMEM((1,), jnp.float32),
        ],
        compiler_params=pltpu.CompilerParams(
            dimension_semantics=("arbitrary", "arbitrary"),
        ),
    )(flat, W)
    return idx3.reshape(M), loss.reshape(())


_NC = 2
_NS = 16
_NW = _NC * _NS
_BPW = M // _NW
_CH = 128
_NCHUNK = _BPW // _CH


def _sc_gather(W, idx):
    idx3 = idx.reshape(_NW, _NCHUNK, _CH)
    mesh = plsc.VectorSubcoreMesh(core_axis_name="c", subcore_axis_name="s")

    @functools.partial(
        pl.kernel,
        mesh=mesh,
        out_type=jax.ShapeDtypeStruct((M, D), jnp.float32),
        scratch_types=[
            pltpu.VMEM((_NCHUNK, _CH), jnp.int32),
            pltpu.VMEM((_CH, D), jnp.float32),
            pltpu.VMEM((_CH, D), jnp.float32),
            pltpu.SemaphoreType.DMA,
            pltpu.SemaphoreType.DMA,
        ],
    )
    def gather(w_hbm, idx_hbm, out_hbm, idx_v, buf0, buf1, sem0, sem1):
        wid = lax.axis_index("s") * _NC + lax.axis_index("c")
        base = wid * _BPW
        pltpu.sync_copy(idx_hbm.at[wid], idx_v)
        bufs = (buf0, buf1)
        sems = (sem0, sem1)
        copies = [None, None]
        for c in range(_NCHUNK):
            s = c % 2
            copies[s] = pltpu.async_copy(w_hbm.at[idx_v.at[c]], bufs[s], sems[s])
            if c >= 1:
                p = (c - 1) % 2
                copies[p].wait()
                pltpu.sync_copy(bufs[p],
                                out_hbm.at[pl.ds(base + (c - 1) * _CH, _CH)])
        last = (_NCHUNK - 1) % 2
        copies[last].wait()
        pltpu.sync_copy(bufs[last],
                        out_hbm.at[pl.ds(base + (_NCHUNK - 1) * _CH, _CH)])

    return gather(W, idx3)


def kernel(feats, W):
    Bb, Ll, Dd = feats.shape
    flat = feats.reshape(M, D)
    idx, loss = _dist_argmin(flat, W)
    quant = _sc_gather(W, idx)
    return (quant.reshape(Bb, Ll, Dd), idx.reshape(Bb, Ll), loss)


# speedup vs baseline: 1.2630x; 1.0574x over previous
import functools

import jax
import jax.numpy as jnp
from jax import lax
from jax.experimental import pallas as pl
from jax.experimental.pallas import tpu as pltpu
from jax.experimental.pallas import tpu_sc as plsc

K = 8192
D = 256
M = 16384

BM = 1024
BK = 1024
NM = M // BM
NK = K // BK


def _wnorm_body(w_ref, wn_ref):
    w = w_ref[...]
    wn_ref[0, :, :] = jnp.sum(w * w, axis=1, keepdims=True)


def _wnorm(W):
    return pl.pallas_call(
        _wnorm_body,
        grid=(NK,),
        in_specs=[pl.BlockSpec((BK, D), lambda k: (k, 0))],
        out_specs=pl.BlockSpec((1, BK, 1), lambda k: (k, 0, 0)),
        out_shape=jax.ShapeDtypeStruct((NK, BK, 1), jnp.float32),
    )(W)


def _fprep_body(f_ref, f2_ref, fn_ref):
    f = f_ref[...]
    f2_ref[...] = f * (-2.0)
    fn_ref[0, 0, :] = jnp.sum(f * f, axis=1)


def _fprep(flat):
    return pl.pallas_call(
        _fprep_body,
        grid=(NM,),
        in_specs=[pl.BlockSpec((BM, D), lambda m: (m, 0))],
        out_specs=[
            pl.BlockSpec((BM, D), lambda m: (m, 0)),
            pl.BlockSpec((1, 1, BM), lambda m: (m, 0, 0)),
        ],
        out_shape=[
            jax.ShapeDtypeStruct((M, D), jnp.float32),
            jax.ShapeDtypeStruct((NM, 1, BM), jnp.float32),
        ],
    )(flat)


def _dist_argmin_body(f2_ref, w_ref, wn_ref, fn_ref, iota_ref,
                      idx_ref, loss_ref, runmin, runidx, acc):
    m = pl.program_id(0)
    k = pl.program_id(1)

    @pl.when(k == 0)
    def _init():
        runmin[...] = jnp.full((1, BM), jnp.inf, jnp.float32)
        runidx[...] = jnp.zeros((1, BM), jnp.float32)

    d2 = lax.dot_general(w_ref[...], f2_ref[...], (((1,), (1,)), ((), ())),
                         preferred_element_type=jnp.float32)
    s = (fn_ref[0, :, :] + d2) + wn_ref[0, :, :]
    colmin = jnp.min(s, axis=0, keepdims=True)
    cand = jnp.where(s == colmin, iota_ref[0, :, :], jnp.float32(K))
    candidx = jnp.min(cand, axis=0, keepdims=True) + (k * BK).astype(jnp.float32)
    better = colmin < runmin[...]
    runidx[...] = jnp.where(better, candidx, runidx[...])
    runmin[...] = jnp.where(better, colmin, runmin[...])

    @pl.when(k == NK - 1)
    def _finish():
        idx_ref[0, 0, :] = runidx[0, :].astype(jnp.int32)

        @pl.when(m == 0)
        def _zero():
            acc[0] = 0.0

        acc[0] += jnp.sum(runmin[...])

        @pl.when(m == NM - 1)
        def _loss():
            loss_ref[...] = jnp.full((1, 1), acc[0] * (1.25 / (M * D)),
                                     jnp.float32)


def _dist_argmin(flat, W):
    wn = _wnorm(W)
    f2, fn = _fprep(flat)
    iotac = jnp.arange(BK, dtype=jnp.float32).reshape(1, BK, 1)
    idx3, loss = pl.pallas_call(
        _dist_argmin_body,
        grid=(NM, NK),
        in_specs=[
            pl.BlockSpec((BM, D), lambda m, k: (m, 0)),
            pl.BlockSpec((BK, D), lambda m, k: (k, 0)),
            pl.BlockSpec((1, BK, 1), lambda m, k: (k, 0, 0)),
            pl.BlockSpec((1, 1, BM), lambda m, k: (m, 0, 0)),
            pl.BlockSpec((1, BK, 1), lambda m, k: (0, 0, 0)),
        ],
        out_specs=[
            pl.BlockSpec((1, 1, BM), lambda m, k: (m, 0, 0)),
            pl.BlockSpec((1, 1), lambda m, k: (0, 0)),
        ],
        out_shape=[
            jax.ShapeDtypeStruct((NM, 1, BM), jnp.int32),
            jax.ShapeDtypeStruct((1, 1), jnp.float32),
        ],
        scratch_shapes=[
            pltpu.VMEM((1, BM), jnp.float32),
            pltpu.VMEM((1, BM), jnp.float32),
            pltpu.SMEM((1,), jnp.float32),
        ],
        compiler_params=pltpu.CompilerParams(
            dimension_semantics=("arbitrary", "arbitrary"),
        ),
    )(f2, W, wn, fn, iotac)
    return idx3.reshape(M), loss.reshape(())


_NC = 2
_NS = 16
_NW = _NC * _NS
_BPW = M // _NW
_CH = 128
_NCHUNK = _BPW // _CH


def _sc_gather(W, idx):
    idx3 = idx.reshape(_NW, _NCHUNK, _CH)
    mesh = plsc.VectorSubcoreMesh(core_axis_name="c", subcore_axis_name="s")

    @functools.partial(
        pl.kernel,
        mesh=mesh,
        out_type=jax.ShapeDtypeStruct((M, D), jnp.float32),
        scratch_types=[
            pltpu.VMEM((_NCHUNK, _CH), jnp.int32),
            pltpu.VMEM((_CH, D), jnp.float32),
            pltpu.VMEM((_CH, D), jnp.float32),
            pltpu.SemaphoreType.DMA,
            pltpu.SemaphoreType.DMA,
        ],
    )
    def gather(w_hbm, idx_hbm, out_hbm, idx_v, buf0, buf1, sem0, sem1):
        wid = lax.axis_index("s") * _NC + lax.axis_index("c")
        base = wid * _BPW
        pltpu.sync_copy(idx_hbm.at[wid], idx_v)
        bufs = (buf0, buf1)
        sems = (sem0, sem1)
        copies = [None, None]
        for c in range(_NCHUNK):
            sl = c % 2
            copies[sl] = pltpu.async_copy(w_hbm.at[idx_v.at[c]], bufs[sl],
                                          sems[sl])
            if c >= 1:
                p = (c - 1) % 2
                copies[p].wait()
                pltpu.sync_copy(bufs[p],
                                out_hbm.at[pl.ds(base + (c - 1) * _CH, _CH)])
        last = (_NCHUNK - 1) % 2
        copies[last].wait()
        pltpu.sync_copy(bufs[last],
                        out_hbm.at[pl.ds(base + (_NCHUNK - 1) * _CH, _CH)])

    return gather(W, idx3)


def kernel(feats, W):
    Bb, Ll, Dd = feats.shape
    flat = feats.reshape(M, D)
    idx, loss = _dist_argmin(flat, W)
    quant = _sc_gather(W, idx)
    return (quant.reshape(Bb, Ll, Dd), idx.reshape(Bb, Ll), loss)
